# baseline (speedup 1.0000x reference)
import functools

import jax
import jax.numpy as jnp
from jax import lax
from jax.experimental import pallas as pl
from jax.experimental.pallas import tpu as pltpu
from jax.experimental.pallas import tpu_sc as plsc

N = 10000
E = 320000
D = 128
L = 5

NC = 2
NS = 16
NW = NC * NS
EPT = E // NW
CH = 125
NCHUNK = EPT // CH
NPAD = 10240
RPT = NPAD // NS
HALF = NCHUNK // 2

@functools.cache
def _make_seg_sum():
    mesh = plsc.VectorSubcoreMesh(core_axis_name="c", subcore_axis_name="s")
    return functools.partial(
        pl.kernel,
        mesh=mesh,
        out_type=jax.ShapeDtypeStruct((NC, NPAD, D), jnp.float32),
        scratch_types=[
            pltpu.VMEM((HALF, CH), jnp.int32),
            pltpu.VMEM((HALF, CH), jnp.int32),
            pltpu.VMEM((2, CH, D), jnp.float32),
            pltpu.VMEM_SHARED((NPAD, D), jnp.float32),
            pltpu.SemaphoreType.DMA,
            pltpu.SemaphoreType.DMA,
            pltpu.SemaphoreType.DMA,
            pltpu.SemaphoreType.DMA,
        ],
    )(_seg_sum_body)


def _seg_sum_body(src_hbm, dst_hbm, h_hbm, zeros_hbm, out_hbm,
                  src_v, dst_v, rows_v, agg_s, gsem0, gsem1, ssem0, ssem1):
    c = lax.axis_index("c")
    s = lax.axis_index("s")
    tile = c * NS + s
    base = tile * NCHUNK

    pltpu.sync_copy(zeros_hbm, agg_s.at[pl.ds(s * RPT, RPT)])

    pltpu.sync_copy(src_hbm.at[pl.ds(base, HALF)], src_v)
    pltpu.sync_copy(dst_hbm.at[pl.ds(base, HALF)], dst_v)

    plsc.subcore_barrier()

    def pair(i, _):
        j0 = 2 * i
        j1 = j0 + 1
        pltpu.make_async_copy(
            h_hbm.at[src_v.at[j0]], rows_v.at[0], gsem0).wait()
        pltpu.async_copy(rows_v.at[0], agg_s.at[dst_v.at[j0]], ssem0, add=True)
        pltpu.make_async_copy(
            h_hbm.at[src_v.at[j1]], rows_v.at[1], gsem1).wait()
        pltpu.async_copy(rows_v.at[1], agg_s.at[dst_v.at[j1]], ssem1, add=True)

        @pl.when(j0 + 2 < HALF)
        def _():
            pltpu.make_async_copy(
                rows_v.at[0], agg_s.at[dst_v.at[j0]], ssem0).wait()
            pltpu.make_async_copy(
                h_hbm.at[src_v.at[j0 + 2]], rows_v.at[0], gsem0).start()
            pltpu.make_async_copy(
                rows_v.at[1], agg_s.at[dst_v.at[j1]], ssem1).wait()
            pltpu.make_async_copy(
                h_hbm.at[src_v.at[j1 + 2]], rows_v.at[1], gsem1).start()
        return ()

    for half in range(2):
        pltpu.make_async_copy(
            h_hbm.at[src_v.at[0]], rows_v.at[0], gsem0).start()
        pltpu.make_async_copy(
            h_hbm.at[src_v.at[1]], rows_v.at[1], gsem1).start()
        lax.fori_loop(0, HALF // 2, pair, ())
        pltpu.make_async_copy(
            rows_v.at[0], agg_s.at[dst_v.at[0]], ssem0).wait()
        pltpu.make_async_copy(
            rows_v.at[1], agg_s.at[dst_v.at[1]], ssem1).wait()
        if half == 0:
            pltpu.sync_copy(src_hbm.at[pl.ds(base + HALF, HALF)], src_v)
            pltpu.sync_copy(dst_hbm.at[pl.ds(base + HALF, HALF)], dst_v)

    plsc.subcore_barrier()
    pltpu.sync_copy(agg_s.at[pl.ds(s * RPT, RPT)],
                    out_hbm.at[c, pl.ds(s * RPT, RPT)])


def _mlp_body(bn, h_ref, a_ref, w1_ref, b1_ref, w2_ref, b2_ref,
              g_ref, be_ref, o_ref):
    z = h_ref[...] + a_ref[0, :N] + a_ref[1, :N]
    z = jnp.maximum(
        jnp.dot(z, w1_ref[...], preferred_element_type=jnp.float32)
        + b1_ref[...], 0.0)
    z = jnp.maximum(
        jnp.dot(z, w2_ref[...], preferred_element_type=jnp.float32)
        + b2_ref[...], 0.0)
    if bn:
        mu = jnp.mean(z, axis=0, keepdims=True)
        var = jnp.mean((z - mu) ** 2, axis=0, keepdims=True)
        z = (z - mu) / jnp.sqrt(var + 1e-5) * g_ref[...] + be_ref[...]
    o_ref[...] = z


def _mlp(bn, h, agg, w1, b1, w2, b2, g, be):
    return pl.pallas_call(
        functools.partial(_mlp_body, bn),
        out_shape=jax.ShapeDtypeStruct((N, D), jnp.float32),
    )(h, agg, w1, b1, w2, b2, g, be)


def kernel(x, edge_index, batch, W1, b1, W2, b2, gamma, beta):
    src = edge_index[0].reshape(NW * NCHUNK, CH)
    dst = edge_index[1].reshape(NW * NCHUNK, CH)
    zeros = jnp.zeros((RPT, D), jnp.float32)
    h = x
    for i in range(L):
        parts = _make_seg_sum()(src, dst, h, zeros)
        h = _mlp(i < L - 1, h, parts, W1[i], b1[i], W2[i], b2[i],
                 gamma[i], beta[i])
    return h


# speedup vs baseline: 1.2910x; 1.2910x over previous
import functools

import jax
import jax.numpy as jnp
from jax import lax
from jax.experimental import pallas as pl
from jax.experimental.pallas import tpu as pltpu
from jax.experimental.pallas import tpu_sc as plsc

N = 10000
E = 320000
D = 128
L = 5

NC = 2
NS = 16
NW = NC * NS
EPT = E // NW
CH = 125
NCHUNK = EPT // CH
NPAD = 10240
RPT = NPAD // NS
HALF = NCHUNK // 2

@functools.cache
def _make_seg_sum():
    mesh = plsc.VectorSubcoreMesh(core_axis_name="c", subcore_axis_name="s")
    return functools.partial(
        pl.kernel,
        mesh=mesh,
        out_type=jax.ShapeDtypeStruct((NC, NPAD, D), jnp.float32),
        scratch_types=[
            pltpu.VMEM((HALF, CH), jnp.int32),
            pltpu.VMEM((HALF, CH), jnp.int32),
            pltpu.VMEM((2, CH, D), jnp.float32),
            pltpu.VMEM_SHARED((NPAD, D), jnp.float32),
            pltpu.SemaphoreType.DMA,
            pltpu.SemaphoreType.DMA,
        ],
    )(_seg_sum_body)


def _seg_sum_body(src_hbm, dst_hbm, h_hbm, zeros_hbm, out_hbm,
                  src_v, dst_v, rows_v, agg_s, gsem0, gsem1):
    c = lax.axis_index("c")
    s = lax.axis_index("s")
    tile = c * NS + s
    base = tile * NCHUNK

    zc = pltpu.make_async_copy(zeros_hbm, agg_s.at[pl.ds(s * RPT, RPT)], gsem0)
    zc.start()
    sc_ = pltpu.make_async_copy(src_hbm.at[pl.ds(base, HALF)], src_v, gsem1)
    sc_.start()
    dc = pltpu.make_async_copy(dst_hbm.at[pl.ds(base, HALF)], dst_v, gsem1)
    dc.start()
    zc.wait()
    sc_.wait()
    dc.wait()

    plsc.subcore_barrier()

    def pair(i, _):
        j0 = 2 * i
        j1 = j0 + 1
        pltpu.make_async_copy(
            h_hbm.at[src_v.at[j1]], rows_v.at[1], gsem1).start()
        pltpu.make_async_copy(
            h_hbm.at[src_v.at[j0]], rows_v.at[0], gsem0).wait()
        pltpu.sync_copy(rows_v.at[0], agg_s.at[dst_v.at[j0]], add=True)

        @pl.when(j0 + 2 < HALF)
        def _():
            pltpu.make_async_copy(
                h_hbm.at[src_v.at[j0 + 2]], rows_v.at[0], gsem0).start()

        pltpu.make_async_copy(
            h_hbm.at[src_v.at[j1]], rows_v.at[1], gsem1).wait()
        pltpu.sync_copy(rows_v.at[1], agg_s.at[dst_v.at[j1]], add=True)
        return ()

    for half in range(2):
        pltpu.make_async_copy(
            h_hbm.at[src_v.at[0]], rows_v.at[0], gsem0).start()
        lax.fori_loop(0, HALF // 2, pair, ())
        if half == 0:
            pltpu.sync_copy(src_hbm.at[pl.ds(base + HALF, HALF)], src_v)
            pltpu.sync_copy(dst_hbm.at[pl.ds(base + HALF, HALF)], dst_v)

    plsc.subcore_barrier()
    pltpu.sync_copy(agg_s.at[pl.ds(s * RPT, RPT)],
                    out_hbm.at[c, pl.ds(s * RPT, RPT)])


def _mlp_body(bn, h_ref, a_ref, w1_ref, b1_ref, w2_ref, b2_ref,
              g_ref, be_ref, o_ref):
    z = h_ref[...] + a_ref[0, :N] + a_ref[1, :N]
    z = jnp.maximum(
        jnp.dot(z, w1_ref[...], preferred_element_type=jnp.float32)
        + b1_ref[...], 0.0)
    z = jnp.maximum(
        jnp.dot(z, w2_ref[...], preferred_element_type=jnp.float32)
        + b2_ref[...], 0.0)
    if bn:
        mu = jnp.mean(z, axis=0, keepdims=True)
        var = jnp.mean((z - mu) ** 2, axis=0, keepdims=True)
        z = (z - mu) / jnp.sqrt(var + 1e-5) * g_ref[...] + be_ref[...]
    o_ref[...] = z


def _mlp(bn, h, agg, w1, b1, w2, b2, g, be):
    return pl.pallas_call(
        functools.partial(_mlp_body, bn),
        out_shape=jax.ShapeDtypeStruct((N, D), jnp.float32),
    )(h, agg, w1, b1, w2, b2, g, be)


def kernel(x, edge_index, batch, W1, b1, W2, b2, gamma, beta):
    src = edge_index[0].reshape(NW * NCHUNK, CH)
    dst = edge_index[1].reshape(NW * NCHUNK, CH)
    zeros = jnp.zeros((RPT, D), jnp.float32)
    h = x
    for i in range(L):
        parts = _make_seg_sum()(src, dst, h, zeros)
        h = _mlp(i < L - 1, h, parts, W1[i], b1[i], W2[i], b2[i],
                 gamma[i], beta[i])
    return h
